# Initial kernel scaffold; baseline (speedup 1.0000x reference)
import jax
import jax.numpy as jnp
from jax.experimental import pallas as pl


def kernel(node_feats, node_attrs, edge_feats, edge_attrs, edge_index, W_up, W_r0, W_r1, W_r2, W_r3, W_lin, W_skip):
    raise NotImplementedError("write your pallas kernel here")


# speedup vs baseline: 1.4889x; 1.4889x over previous
import functools
import math

import jax
import jax.numpy as jnp
from jax import lax
from jax.experimental import pallas as pl
from jax.experimental.pallas import tpu as pltpu
from jax.experimental.pallas import tpu_sc as plsc

N = 10000
E = 320000
D = 128
NUM_ELEM = 10
NUM_BESSEL = 8
HIDDEN = 64
AVG_NEIGH = 32.0
_SILU_NORM = 1.6790532

NC = 2
NS = 16
L = 16
NW = NC * NS

K = 128
CPW = 80
CHUNKS = NW * CPW
EP = CHUNKS * K
NP = 10240
RPT = NP // NS


def _silu(x):
    return x * jax.nn.sigmoid(x) * _SILU_NORM


def _h_body(nf_ref, wup_ref, h_ref):
    h_ref[...] = jnp.dot(nf_ref[...], wup_ref[...],
                         preferred_element_type=jnp.float32) * (1.0 / math.sqrt(D))


def _node_linear(node_feats, W_up):
    BN = 2000
    return pl.pallas_call(
        _h_body,
        out_shape=jax.ShapeDtypeStruct((N, D), jnp.float32),
        grid=(N // BN,),
        in_specs=[pl.BlockSpec((BN, D), lambda i: (i, 0)),
                  pl.BlockSpec((D, D), lambda i: (0, 0))],
        out_specs=pl.BlockSpec((BN, D), lambda i: (i, 0)),
    )(node_feats, W_up)


def _coeff_body(ef_ref, ea_ref, w0_ref, w1_ref, w2_ref, w3_ref, out_ref):
    x = jnp.dot(ef_ref[...], w0_ref[...],
                preferred_element_type=jnp.float32) * (1.0 / math.sqrt(NUM_BESSEL))
    x = _silu(x)
    x = jnp.dot(x, w1_ref[...],
                preferred_element_type=jnp.float32) * (1.0 / math.sqrt(HIDDEN))
    x = _silu(x)
    x = jnp.dot(x, w2_ref[...],
                preferred_element_type=jnp.float32) * (1.0 / math.sqrt(HIDDEN))
    x = _silu(x)
    tw = jnp.dot(x, w3_ref[...],
                 preferred_element_type=jnp.float32) * (1.0 / math.sqrt(HIDDEN))
    out_ref[...] = tw * ea_ref[...]


def _edge_coeff(edge_feats_p, edge_attrs_p, W_r0, W_r1, W_r2, W_r3):
    BE = 2048
    return pl.pallas_call(
        _coeff_body,
        out_shape=jax.ShapeDtypeStruct((EP, D), jnp.float32),
        grid=(EP // BE,),
        in_specs=[pl.BlockSpec((BE, NUM_BESSEL), lambda i: (i, 0)),
                  pl.BlockSpec((BE, 1), lambda i: (i, 0)),
                  pl.BlockSpec((NUM_BESSEL, HIDDEN), lambda i: (0, 0)),
                  pl.BlockSpec((HIDDEN, HIDDEN), lambda i: (0, 0)),
                  pl.BlockSpec((HIDDEN, HIDDEN), lambda i: (0, 0)),
                  pl.BlockSpec((HIDDEN, D), lambda i: (0, 0))],
        out_specs=pl.BlockSpec((BE, D), lambda i: (i, 0)),
    )(edge_feats_p, edge_attrs_p, W_r0, W_r1, W_r2, W_r3)


def _sc_body(h_hbm, coeff_hbm, src_hbm, dst_hbm, out_hbm,
             src_v, dst_v, rows_v, coeff_v, msg_sh, sem):
    c = lax.axis_index("c")
    s = lax.axis_index("s")
    wid = s * NC + c

    zero = jnp.zeros((L,), jnp.float32)

    def zrow(r, carry):
        for j in range(D // L):
            rows_v[r, pl.ds(j * L, L)] = zero
        return carry

    lax.fori_loop(0, K, zrow, 0)
    base = s * RPT
    for t in range(RPT // K):
        pltpu.sync_copy(rows_v, msg_sh.at[pl.ds(base + t * K, K)])
    plsc.subcore_barrier()

    def chunk_body(i, carry):
        chunk = wid * CPW + i
        pltpu.sync_copy(src_hbm.at[chunk], src_v)
        pltpu.sync_copy(dst_hbm.at[chunk], dst_v.at[0])
        pltpu.async_copy(h_hbm.at[src_v], rows_v, sem).wait()
        pltpu.sync_copy(coeff_hbm.at[pl.ds(chunk * K, K)], coeff_v)

        def mrow(r, cc):
            for j in range(D // L):
                sl = pl.ds(j * L, L)
                rows_v[r, sl] = rows_v[r, sl] * coeff_v[r, sl]
            return cc

        lax.fori_loop(0, K, mrow, 0)
        pltpu.sync_copy(rows_v, msg_sh.at[dst_v.at[0]], add=True)
        return carry

    lax.fori_loop(0, CPW, chunk_body, 0)
    plsc.subcore_barrier()
    pltpu.sync_copy(msg_sh.at[pl.ds(base, RPT)], out_hbm.at[c, pl.ds(base, RPT)])


def _sc_scatter(h, coeff_p, src_p, dst_p):
    mesh = plsc.VectorSubcoreMesh(core_axis_name="c", subcore_axis_name="s",
                                  num_cores=NC, num_subcores=NS)
    fn = pl.kernel(
        _sc_body,
        out_type=jax.ShapeDtypeStruct((NC, NP, D), jnp.float32),
        mesh=mesh,
        scratch_types=[
            pltpu.VMEM((K,), jnp.int32),
            pltpu.VMEM((1, K), jnp.int32),
            pltpu.VMEM((K, D), jnp.float32),
            pltpu.VMEM((K, D), jnp.float32),
            pltpu.VMEM_SHARED((NP, D), jnp.float32),
            pltpu.SemaphoreType.DMA,
        ],
    )
    return fn(h, coeff_p, src_p, dst_p)


def _out_body(part_ref, attrs_ref, wlin_ref, wskt_ref, out_ref):
    m = part_ref[0] + part_ref[1]
    m2 = jnp.dot(m, wlin_ref[...], preferred_element_type=jnp.float32) * (
        1.0 / (math.sqrt(D) * AVG_NEIGH))
    attrs = attrs_ref[...]
    acc = attrs[:, 0][:, None] * jnp.dot(m2, wskt_ref[0],
                                         preferred_element_type=jnp.float32)
    for j in range(1, NUM_ELEM):
        acc = acc + attrs[:, j][:, None] * jnp.dot(
            m2, wskt_ref[j], preferred_element_type=jnp.float32)
    out_ref[...] = acc * (1.0 / math.sqrt(D * NUM_ELEM))


def _final(partials, node_attrs, W_lin, W_skip_t):
    BN = 2000
    return pl.pallas_call(
        _out_body,
        out_shape=jax.ShapeDtypeStruct((N, D), jnp.float32),
        grid=(N // BN,),
        in_specs=[pl.BlockSpec((NC, BN, D), lambda i: (0, i, 0)),
                  pl.BlockSpec((BN, NUM_ELEM), lambda i: (i, 0)),
                  pl.BlockSpec((D, D), lambda i: (0, 0)),
                  pl.BlockSpec((NUM_ELEM, D, D), lambda i: (0, 0, 0))],
        out_specs=pl.BlockSpec((BN, D), lambda i: (i, 0)),
    )(partials, node_attrs, W_lin, W_skip_t)


def kernel(node_feats, node_attrs, edge_feats, edge_attrs, edge_index,
           W_up, W_r0, W_r1, W_r2, W_r3, W_lin, W_skip):
    pad = EP - E
    ef_p = jnp.pad(edge_feats, ((0, pad), (0, 0)))
    ea_p = jnp.pad(edge_attrs, ((0, pad), (0, 0)))
    src_p = jnp.pad(edge_index[0], (0, pad)).reshape(CHUNKS, K)
    dst_p = jnp.pad(edge_index[1], (0, pad)).reshape(CHUNKS, K)

    h = _node_linear(node_feats, W_up)
    coeff = _edge_coeff(ef_p, ea_p, W_r0, W_r1, W_r2, W_r3)
    partials = _sc_scatter(h, coeff, src_p, dst_p)
    return _final(partials, node_attrs, W_lin, W_skip.transpose(1, 0, 2))


# speedup vs baseline: 2.6780x; 1.7986x over previous
import functools
import math

import jax
import jax.numpy as jnp
from jax import lax
from jax.experimental import pallas as pl
from jax.experimental.pallas import tpu as pltpu
from jax.experimental.pallas import tpu_sc as plsc

N = 10000
E = 320000
D = 128
NUM_ELEM = 10
NUM_BESSEL = 8
HIDDEN = 64
AVG_NEIGH = 32.0
_SILU_NORM = 1.6790532

NC = 2
NS = 16
L = 16
NW = NC * NS

K = 80
CHUNKS = E // K
CPW = CHUNKS // NW
NP = 10240
RPT = NP // NS


def _silu(x):
    return x * jax.nn.sigmoid(x) * _SILU_NORM


def _h_body(nf_ref, wup_ref, h_ref):
    h_ref[...] = jnp.dot(nf_ref[...], wup_ref[...],
                         preferred_element_type=jnp.float32) * (1.0 / math.sqrt(D))


def _node_linear(node_feats, W_up):
    BN = 2000
    return pl.pallas_call(
        _h_body,
        out_shape=jax.ShapeDtypeStruct((N, D), jnp.float32),
        grid=(N // BN,),
        in_specs=[pl.BlockSpec((BN, D), lambda i: (i, 0)),
                  pl.BlockSpec((D, D), lambda i: (0, 0))],
        out_specs=pl.BlockSpec((BN, D), lambda i: (i, 0)),
    )(node_feats, W_up)


def _coeff_body(ef_ref, ea_ref, w0_ref, w1_ref, w2_ref, w3_ref, out_ref):
    x = jnp.dot(ef_ref[...], w0_ref[...],
                preferred_element_type=jnp.float32) * (1.0 / math.sqrt(NUM_BESSEL))
    x = _silu(x)
    x = jnp.dot(x, w1_ref[...],
                preferred_element_type=jnp.float32) * (1.0 / math.sqrt(HIDDEN))
    x = _silu(x)
    x = jnp.dot(x, w2_ref[...],
                preferred_element_type=jnp.float32) * (1.0 / math.sqrt(HIDDEN))
    x = _silu(x)
    tw = jnp.dot(x, w3_ref[...],
                 preferred_element_type=jnp.float32) * (1.0 / math.sqrt(HIDDEN))
    out_ref[...] = tw * ea_ref[...]


def _edge_coeff(edge_feats_p, edge_attrs_p, W_r0, W_r1, W_r2, W_r3):
    BE = 2000
    return pl.pallas_call(
        _coeff_body,
        out_shape=jax.ShapeDtypeStruct((E, D), jnp.float32),
        grid=(E // BE,),
        in_specs=[pl.BlockSpec((BE, NUM_BESSEL), lambda i: (i, 0)),
                  pl.BlockSpec((BE, 1), lambda i: (i, 0)),
                  pl.BlockSpec((NUM_BESSEL, HIDDEN), lambda i: (0, 0)),
                  pl.BlockSpec((HIDDEN, HIDDEN), lambda i: (0, 0)),
                  pl.BlockSpec((HIDDEN, HIDDEN), lambda i: (0, 0)),
                  pl.BlockSpec((HIDDEN, D), lambda i: (0, 0))],
        out_specs=pl.BlockSpec((BE, D), lambda i: (i, 0)),
    )(edge_feats_p, edge_attrs_p, W_r0, W_r1, W_r2, W_r3)


def _sc_body(h_hbm, coeff_hbm, src_hbm, dst_hbm, out_hbm,
             src0_v, src1_v, dst0_v, dst1_v, rows0_v, rows1_v,
             coef0_v, coef1_v, msg_sh,
             sg0, sg1, sc0, sc1, ss0, ss1):
    c = lax.axis_index("c")
    s = lax.axis_index("s")
    wid = s * NC + c
    bufs = ((src0_v, dst0_v, rows0_v, coef0_v, sg0, sc0, ss0),
            (src1_v, dst1_v, rows1_v, coef1_v, sg1, sc1, ss1))

    zero = jnp.zeros((L,), jnp.float32)

    def zrow(r, carry):
        for j in range(D // L):
            rows0_v[r, pl.ds(j * L, L)] = zero
        return carry

    lax.fori_loop(0, K, zrow, 0)
    base = s * RPT
    for t in range(RPT // K):
        pltpu.sync_copy(rows0_v, msg_sh.at[pl.ds(base + t * K, K)])
    plsc.subcore_barrier()

    def _mult(rows, coef):
        def mrow(r):
            for j in range(D // L):
                sl = pl.ds(j * L, L)
                rows[r, sl] = rows[r, sl] * coef[r, sl]

        plsc.parallel_loop(0, K, 1, unroll=2)(mrow)

    def body(g, carry):
        for b in (0, 1):
            src_v, dst_v, rows, coef, sg, sc_, ss = bufs[b]

            @pl.when(g >= 1)
            def _():
                pltpu.make_async_copy(rows, msg_sh.at[dst_v.at[0]], ss).wait()

            chunk = wid + (2 * g + b) * NW
            pltpu.sync_copy(src_hbm.at[chunk], src_v)
            pltpu.sync_copy(dst_hbm.at[chunk], dst_v.at[0])
            pltpu.async_copy(h_hbm.at[src_v], rows, sg)
            pltpu.async_copy(coeff_hbm.at[pl.ds(chunk * K, K)], coef, sc_)
        for b in (0, 1):
            src_v, dst_v, rows, coef, sg, sc_, ss = bufs[b]
            chunk = wid + (2 * g + b) * NW
            pltpu.make_async_copy(h_hbm.at[src_v], rows, sg).wait()
            pltpu.make_async_copy(coeff_hbm.at[pl.ds(chunk * K, K)], coef,
                                  sc_).wait()
            _mult(rows, coef)
            pltpu.async_copy(rows, msg_sh.at[dst_v.at[0]], ss, add=True)
        return carry

    lax.fori_loop(0, CPW // 2, body, 0)
    for b in (0, 1):
        src_v, dst_v, rows, coef, sg, sc_, ss = bufs[b]
        pltpu.make_async_copy(rows, msg_sh.at[dst_v.at[0]], ss).wait()

    tchunk = wid + (CPW - 1) * NW
    pltpu.sync_copy(src_hbm.at[tchunk], src0_v)
    pltpu.sync_copy(dst_hbm.at[tchunk], dst0_v.at[0])
    pltpu.async_copy(h_hbm.at[src0_v], rows0_v, sg0).wait()
    pltpu.async_copy(coeff_hbm.at[pl.ds(tchunk * K, K)], coef0_v, sc0).wait()
    _mult(rows0_v, coef0_v)
    pltpu.async_copy(rows0_v, msg_sh.at[dst0_v.at[0]], ss0, add=True).wait()

    plsc.subcore_barrier()
    pltpu.sync_copy(msg_sh.at[pl.ds(base, RPT)], out_hbm.at[c, pl.ds(base, RPT)])


def _sc_scatter(h, coeff_p, src_p, dst_p):
    mesh = plsc.VectorSubcoreMesh(core_axis_name="c", subcore_axis_name="s",
                                  num_cores=NC, num_subcores=NS)
    fn = pl.kernel(
        _sc_body,
        out_type=jax.ShapeDtypeStruct((NC, NP, D), jnp.float32),
        mesh=mesh,
        scratch_types=[
            pltpu.VMEM((K,), jnp.int32),
            pltpu.VMEM((K,), jnp.int32),
            pltpu.VMEM((1, K), jnp.int32),
            pltpu.VMEM((1, K), jnp.int32),
            pltpu.VMEM((K, D), jnp.float32),
            pltpu.VMEM((K, D), jnp.float32),
            pltpu.VMEM((K, D), jnp.float32),
            pltpu.VMEM((K, D), jnp.float32),
            pltpu.VMEM_SHARED((NP, D), jnp.float32),
            pltpu.SemaphoreType.DMA,
            pltpu.SemaphoreType.DMA,
            pltpu.SemaphoreType.DMA,
            pltpu.SemaphoreType.DMA,
            pltpu.SemaphoreType.DMA,
            pltpu.SemaphoreType.DMA,
        ],
    )
    return fn(h, coeff_p, src_p, dst_p)


def _out_body(part_ref, attrs_ref, wlin_ref, wskt_ref, out_ref):
    m = part_ref[0] + part_ref[1]
    m2 = jnp.dot(m, wlin_ref[...], preferred_element_type=jnp.float32) * (
        1.0 / (math.sqrt(D) * AVG_NEIGH))
    attrs = attrs_ref[...]
    acc = attrs[:, 0][:, None] * jnp.dot(m2, wskt_ref[0],
                                         preferred_element_type=jnp.float32)
    for j in range(1, NUM_ELEM):
        acc = acc + attrs[:, j][:, None] * jnp.dot(
            m2, wskt_ref[j], preferred_element_type=jnp.float32)
    out_ref[...] = acc * (1.0 / math.sqrt(D * NUM_ELEM))


def _final(partials, node_attrs, W_lin, W_skip_t):
    BN = 2000
    return pl.pallas_call(
        _out_body,
        out_shape=jax.ShapeDtypeStruct((N, D), jnp.float32),
        grid=(N // BN,),
        in_specs=[pl.BlockSpec((NC, BN, D), lambda i: (0, i, 0)),
                  pl.BlockSpec((BN, NUM_ELEM), lambda i: (i, 0)),
                  pl.BlockSpec((D, D), lambda i: (0, 0)),
                  pl.BlockSpec((NUM_ELEM, D, D), lambda i: (0, 0, 0))],
        out_specs=pl.BlockSpec((BN, D), lambda i: (i, 0)),
    )(partials, node_attrs, W_lin, W_skip_t)


def kernel(node_feats, node_attrs, edge_feats, edge_attrs, edge_index,
           W_up, W_r0, W_r1, W_r2, W_r3, W_lin, W_skip):
    src_p = edge_index[0].reshape(CHUNKS, K)
    dst_p = edge_index[1].reshape(CHUNKS, K)

    h = _node_linear(node_feats, W_up)
    coeff = _edge_coeff(edge_feats, edge_attrs, W_r0, W_r1, W_r2, W_r3)
    partials = _sc_scatter(h, coeff, src_p, dst_p)
    return _final(partials, node_attrs, W_lin, W_skip.transpose(1, 0, 2))
